# baseline (speedup 1.0000x reference)
import functools

import jax
import jax.numpy as jnp
from jax import lax
from jax.experimental import pallas as pl
from jax.experimental.pallas import tpu as pltpu
from jax.experimental.pallas import tpu_sc as plsc

_TAU = 0.007
_B = 8192
_D = 1024
_C = 1024
_BM = 512
_HALF = _B // 2
_TPT = _HALF // 16
_NBLK = _B // _BM
_HBLK = _NBLK // 2


def _maxprob(logits):
    m = jnp.max(logits, axis=-1, keepdims=True)
    un = jnp.exp(logits - m)
    s = jnp.sum(un, axis=-1, keepdims=True)
    return jnp.max(un / s, axis=-1)


def _dot(a, b):
    return jnp.dot(a, b, preferred_element_type=jnp.float32)


def _stage0_body(x_ref, W0_ref, b0_ref, P0_ref, q0_ref,
                 out_ref, h0_ref, mask_ref):
    x = x_ref[...]
    h0 = jnp.maximum(_dot(x, W0_ref[...]) + b0_ref[...], 0.0)
    p0 = _dot(h0, P0_ref[...]) + q0_ref[...]
    exited = _maxprob(p0) > _TAU
    out_ref[...] = p0
    h0_ref[...] = h0
    mask_ref[...] = exited.astype(jnp.int32).reshape(1, 1, _BM)


def _stage0(x, W0, b0, P0, q0):
    full = lambda d0, d1: pl.BlockSpec((d0, d1), lambda i: (0, 0))
    vec = lambda d: pl.BlockSpec((1, d), lambda i: (0, 0))
    return pl.pallas_call(
        _stage0_body,
        grid=(_NBLK,),
        in_specs=[
            pl.BlockSpec((_BM, _D), lambda i: (i, 0)),
            full(_D, _D), vec(_D), full(_D, _C), vec(_C),
        ],
        out_specs=[
            pl.BlockSpec((_BM, _C), lambda i: (i, 0)),
            pl.BlockSpec((_BM, _D), lambda i: (i, 0)),
            pl.BlockSpec((1, 1, _BM), lambda i: (i, 0, 0)),
        ],
        out_shape=[
            jax.ShapeDtypeStruct((_B, _C), jnp.float32),
            jax.ShapeDtypeStruct((_B, _D), jnp.float32),
            jax.ShapeDtypeStruct((_NBLK, 1, _BM), jnp.int32),
        ],
    )(x, W0, b0.reshape(1, _D), P0, q0.reshape(1, _C))


def _compact_body(mask_hbm, idx_hbm, cnt_hbm,
                  mask_v, idxbuf_v, cbuf_v, sem):
    c = lax.axis_index("c")
    s = lax.axis_index("s")
    tok_base = c * _HALF + s * _TPT
    seg_base = c * _HALF

    pltpu.sync_copy(mask_hbm.at[pl.ds(seg_base, _HALF)], mask_v)
    offset = jnp.int32(0)
    total = jnp.int32(0)
    tile_cnt = jnp.int32(0)
    for r in range(16):
        accv = jnp.zeros((16,), jnp.int32)
        for u in range(_TPT // 16):
            mv = mask_v[pl.ds(r * _TPT + u * 16, 16)]
            accv = accv + jnp.where(mv == 0, jnp.int32(1), jnp.int32(0))
        cr = jnp.sum(accv)
        offset = offset + jnp.where(jnp.int32(r) < s, cr, 0)
        total = total + cr
        tile_cnt = tile_cnt + jnp.where(jnp.int32(r) == s, cr, 0)

    base = jnp.int32(0)
    for u in range(_TPT // 16):
        mv = mask_v[pl.ds(s * _TPT + u * 16, 16)]
        surv = mv == 0
        s32 = jnp.where(surv, jnp.int32(1), jnp.int32(0))
        csum = plsc.cumsum(s32)
        pos = base + csum - 1
        ids = tok_base + u * 16 + lax.iota(jnp.int32, 16)
        plsc.store_scatter(idxbuf_v, [pos], ids, mask=surv)
        base = base + jnp.sum(s32)

    for u in range(_TPT // 16):
        j = u * 16 + lax.iota(jnp.int32, 16)
        valid = j < tile_cnt
        tgt = jnp.where(valid, seg_base + offset + j, jnp.int32(_B))
        cp = pltpu.make_async_copy(idxbuf_v.at[pl.ds(u * 16, 16)],
                                   idx_hbm.at[tgt], sem)
        cp.start()
        cp.wait()

    @pl.when(s == 0)
    def _():
        cbuf_v[...] = jnp.full((16,), total, dtype=jnp.int32)
        pltpu.sync_copy(cbuf_v, cnt_hbm.at[c])


def _gather_body(idx_hbm, h0_hbm, hc_hbm, idxchunk_v, row_v, sem):
    c = lax.axis_index("c")
    s = lax.axis_index("s")
    cpos = c * _HALF + s * _TPT
    pltpu.sync_copy(idx_hbm.at[pl.ds(cpos, _TPT)], idxchunk_v)
    for w in range(_TPT // 16):
        idxv = idxchunk_v[pl.ds(w * 16, 16)]
        idxv = jnp.minimum(jnp.maximum(idxv, 0), jnp.int32(_B - 1))
        cp = pltpu.make_async_copy(h0_hbm.at[idxv], row_v, sem)
        cp.start()
        cp.wait()
        pltpu.sync_copy(row_v, hc_hbm.at[pl.ds(cpos + w * 16, 16)])


def _route(mask, h0):
    mesh = plsc.VectorSubcoreMesh(core_axis_name="c", subcore_axis_name="s")
    compact = pl.kernel(
        _compact_body,
        mesh=mesh,
        out_type=[
            jax.ShapeDtypeStruct((_B + 16, ), jnp.int32),
            jax.ShapeDtypeStruct((2, 16), jnp.int32),
        ],
        scratch_types=[
            pltpu.VMEM((_HALF,), jnp.int32),
            pltpu.VMEM((_TPT,), jnp.int32),
            pltpu.VMEM((16,), jnp.int32),
            pltpu.SemaphoreType.DMA,
        ],
        compiler_params=pltpu.CompilerParams(needs_layout_passes=False),
    )
    idx1, cnt = compact(mask)
    gather = pl.kernel(
        _gather_body,
        mesh=mesh,
        out_type=jax.ShapeDtypeStruct((_B, _D), jnp.float32),
        scratch_types=[
            pltpu.VMEM((_TPT,), jnp.int32),
            pltpu.VMEM((16, _D), jnp.float32),
            pltpu.SemaphoreType.DMA,
        ],
        compiler_params=pltpu.CompilerParams(needs_layout_passes=False),
    )
    hc = gather(idx1, h0)
    return idx1, cnt, hc


def _stage12_body(cnt_ref, hc_ref, W1_ref, b1_ref, P1_ref, q1_ref,
                  W2_ref, b2_ref, merged_ref):
    i = pl.program_id(0)
    half = i // _HBLK
    n = cnt_ref[half, 0]
    active = (i % _HBLK) * _BM < n

    @pl.when(active)
    def _():
        hc = hc_ref[...]
        h1 = jnp.maximum(_dot(hc, W1_ref[...]) + b1_ref[...], 0.0)
        p1 = _dot(h1, P1_ref[...]) + q1_ref[...]
        m1 = _maxprob(p1) > _TAU
        f = _dot(h1, W2_ref[...]) + b2_ref[...]
        merged_ref[...] = jnp.where(m1[:, None], p1, f)


def _stage12(cnt, hc, W1, b1, P1, q1, W2, b2):
    def hc_map(i, cnt_ref):
        half = i // _HBLK
        active = (i % _HBLK) * _BM < cnt_ref[half, 0]
        return (jnp.where(active, i, 0), 0)

    full = lambda d0, d1: pl.BlockSpec((d0, d1), lambda i, c_: (0, 0))
    vec = lambda d: pl.BlockSpec((1, d), lambda i, c_: (0, 0))
    grid_spec = pltpu.PrefetchScalarGridSpec(
        num_scalar_prefetch=1,
        grid=(_NBLK,),
        in_specs=[
            pl.BlockSpec((_BM, _D), hc_map),
            full(_D, _D), vec(_D), full(_D, _C), vec(_C),
            full(_D, _C), vec(_C),
        ],
        out_specs=pl.BlockSpec((_BM, _C), lambda i, c_: (i, 0)),
    )
    return pl.pallas_call(
        _stage12_body,
        grid_spec=grid_spec,
        out_shape=jax.ShapeDtypeStruct((_B, _C), jnp.float32),
    )(cnt, hc, W1, b1.reshape(1, _D), P1, q1.reshape(1, _C),
      W2, b2.reshape(1, _C))


def _scatter_body(merged_hbm, idx_hbm, cnt_hbm, out_hbm,
                  idxchunk_v, row_v, cbuf_v, sem):
    c = lax.axis_index("c")
    s = lax.axis_index("s")
    seg_base = c * _HALF
    cpos0 = s * _TPT

    pltpu.sync_copy(cnt_hbm.at[c], cbuf_v)
    n = jnp.max(cbuf_v[...])
    pltpu.sync_copy(idx_hbm.at[pl.ds(seg_base + cpos0, _TPT)], idxchunk_v)

    for w in range(_TPT // 16):
        wbase = cpos0 + w * 16
        rem = n - wbase

        @pl.when(rem >= 16)
        def _():
            idxv = idxchunk_v[pl.ds(w * 16, 16)]
            idxv = jnp.minimum(jnp.maximum(idxv, 0), jnp.int32(_B - 1))
            pltpu.sync_copy(merged_hbm.at[pl.ds(seg_base + wbase, 16)], row_v)
            pltpu.make_async_copy(row_v, out_hbm.at[idxv], sem).start()
            pltpu.make_async_copy(row_v, out_hbm.at[idxv], sem).wait()

        @pl.when((rem > 0) & (rem < 16))
        def _():
            idxv = idxchunk_v[pl.ds(w * 16, 16)]
            idxv = jnp.minimum(jnp.maximum(idxv, 0), jnp.int32(_B - 1))
            pltpu.sync_copy(merged_hbm.at[pl.ds(seg_base + wbase, 16)], row_v)
            lanes = lax.iota(jnp.int32, 16)

            def body(j, carry):
                t = jnp.sum(jnp.where(lanes == j, idxv, 0))
                pltpu.sync_copy(row_v.at[pl.ds(j, 1)],
                                out_hbm.at[pl.ds(t, 1)])
                return carry

            lax.fori_loop(0, rem, body, jnp.int32(0))


def _scatter(merged, idx1, cnt, out0):
    mesh = plsc.VectorSubcoreMesh(core_axis_name="c", subcore_axis_name="s")
    k = pl.kernel(
        _scatter_body,
        mesh=mesh,
        out_type=(),
        scratch_types=[
            pltpu.VMEM((_TPT,), jnp.int32),
            pltpu.VMEM((16, _C), jnp.float32),
            pltpu.VMEM((16,), jnp.int32),
            pltpu.SemaphoreType.DMA,
        ],
        compiler_params=pltpu.CompilerParams(needs_layout_passes=False),
    )
    out_ref = jax.new_ref(out0)
    k(merged, idx1, cnt, out_ref)
    return jax.freeze(out_ref)


def kernel(input, W0, b0, W1, b1, W2, b2, P0, q0, P1, q1):
    out0, h0, mask3d = _stage0(input, W0, b0, P0, q0)
    mask = mask3d.reshape(_B)
    idx1, cnt, hc = _route(mask, h0)
    merged = _stage12(cnt, hc, W1, b1, P1, q1, W2, b2)
    return _scatter(merged, idx1, cnt, out0)


# speedup vs baseline: 5.2237x; 5.2237x over previous
import functools

import jax
import jax.numpy as jnp
from jax import lax
from jax.experimental import pallas as pl
from jax.experimental.pallas import tpu as pltpu
from jax.experimental.pallas import tpu_sc as plsc

_TAU = 0.007
_B = 8192
_D = 1024
_C = 1024
_BM = 512
_HALF = _B // 2
_TPT = _HALF // 16
_NBLK = _B // _BM
_HBLK = _NBLK // 2


def _maxprob(logits):
    m = jnp.max(logits, axis=-1, keepdims=True)
    un = jnp.exp(logits - m)
    s = jnp.sum(un, axis=-1, keepdims=True)
    return jnp.max(un / s, axis=-1)


def _dot(a, b):
    return jnp.dot(a, b, preferred_element_type=jnp.float32)


def _stage0_body(x_ref, W0_ref, b0_ref, P0_ref, q0_ref,
                 out_ref, h0_ref, mask_ref):
    x = x_ref[...]
    h0 = jnp.maximum(_dot(x, W0_ref[...]) + b0_ref[...], 0.0)
    p0 = _dot(h0, P0_ref[...]) + q0_ref[...]
    exited = _maxprob(p0) > _TAU
    out_ref[...] = p0
    h0_ref[...] = h0
    mask_ref[...] = exited.astype(jnp.int32).reshape(1, 1, _BM)


def _stage0(x, W0, b0, P0, q0):
    full = lambda d0, d1: pl.BlockSpec((d0, d1), lambda i: (0, 0))
    vec = lambda d: pl.BlockSpec((1, d), lambda i: (0, 0))
    return pl.pallas_call(
        _stage0_body,
        grid=(_NBLK,),
        in_specs=[
            pl.BlockSpec((_BM, _D), lambda i: (i, 0)),
            full(_D, _D), vec(_D), full(_D, _C), vec(_C),
        ],
        out_specs=[
            pl.BlockSpec((_BM, _C), lambda i: (i, 0)),
            pl.BlockSpec((_BM, _D), lambda i: (i, 0)),
            pl.BlockSpec((1, 1, _BM), lambda i: (i, 0, 0)),
        ],
        out_shape=[
            jax.ShapeDtypeStruct((_B, _C), jnp.float32),
            jax.ShapeDtypeStruct((_B, _D), jnp.float32),
            jax.ShapeDtypeStruct((_NBLK, 1, _BM), jnp.int32),
        ],
    )(x, W0, b0.reshape(1, _D), P0, q0.reshape(1, _C))


def _compact_body(mask_hbm, idx_hbm, cnt_hbm,
                  mask_v, idxbuf_v, cbuf_v, sem):
    c = lax.axis_index("c")
    s = lax.axis_index("s")
    tok_base = c * _HALF + s * _TPT
    seg_base = c * _HALF

    pltpu.sync_copy(mask_hbm.at[pl.ds(seg_base, _HALF)], mask_v)
    offset = jnp.int32(0)
    total = jnp.int32(0)
    tile_cnt = jnp.int32(0)
    for r in range(16):
        accv = jnp.zeros((16,), jnp.int32)
        for u in range(_TPT // 16):
            mv = mask_v[pl.ds(r * _TPT + u * 16, 16)]
            accv = accv + jnp.where(mv == 0, jnp.int32(1), jnp.int32(0))
        cr = jnp.sum(accv)
        offset = offset + jnp.where(jnp.int32(r) < s, cr, 0)
        total = total + cr
        tile_cnt = tile_cnt + jnp.where(jnp.int32(r) == s, cr, 0)

    base = jnp.int32(0)
    for u in range(_TPT // 16):
        mv = mask_v[pl.ds(s * _TPT + u * 16, 16)]
        surv = mv == 0
        s32 = jnp.where(surv, jnp.int32(1), jnp.int32(0))
        csum = plsc.cumsum(s32)
        pos = base + csum - 1
        ids = tok_base + u * 16 + lax.iota(jnp.int32, 16)
        plsc.store_scatter(idxbuf_v, [pos], ids, mask=surv)
        base = base + jnp.sum(s32)

    def idtgt(u):
        j = u * 16 + lax.iota(jnp.int32, 16)
        tgt = jnp.where(j < tile_cnt, seg_base + offset + j, jnp.int32(_B))
        return pltpu.make_async_copy(idxbuf_v.at[pl.ds(u * 16, 16)],
                                     idx_hbm.at[tgt], sem)

    for u in range(_TPT // 16):
        @pl.when(u * 16 < tile_cnt)
        def _(u=u):
            idtgt(u).start()
    for u in range(_TPT // 16):
        @pl.when(u * 16 < tile_cnt)
        def _(u=u):
            idtgt(u).wait()

    @pl.when(s == 0)
    def _():
        cbuf_v[...] = jnp.full((16,), total, dtype=jnp.int32)
        pltpu.sync_copy(cbuf_v, cnt_hbm.at[c])


_RING = 6
_NW = _TPT // 16


def _gather_body(cnt_hbm, idx_hbm, h0_hbm, hc_hbm,
                 cnt_v, idxchunk_v, buf_v, *sems):
    gsem, osem = sems[:_RING], sems[_RING:]
    c = lax.axis_index("c")
    s = lax.axis_index("s")
    seg = c * _HALF
    cpos0 = s * _TPT
    pltpu.sync_copy(cnt_hbm.at[c], cnt_v)
    n = jnp.max(cnt_v[...])
    pltpu.sync_copy(idx_hbm.at[pl.ds(seg + cpos0, _TPT)], idxchunk_v)

    def act(w):
        return cpos0 + w * 16 < n

    def g_cp(w):
        idxv = idxchunk_v[pl.ds(w * 16, 16)]
        idxv = jnp.minimum(jnp.maximum(idxv, 0), jnp.int32(_B - 1))
        return pltpu.make_async_copy(h0_hbm.at[idxv], buf_v.at[w % _RING],
                                     gsem[w % _RING])

    def o_cp(w):
        return pltpu.make_async_copy(
            buf_v.at[w % _RING],
            hc_hbm.at[pl.ds(seg + cpos0 + w * 16, 16)], osem[w % _RING])

    for w in range(min(_RING, _NW)):
        @pl.when(act(w))
        def _(w=w):
            g_cp(w).start()
    for w in range(_NW):
        @pl.when(act(w))
        def _(w=w):
            g_cp(w).wait()
            o_cp(w).start()
        if w + _RING < _NW:
            @pl.when(act(w + _RING))
            def _(w=w):
                o_cp(w).wait()
                g_cp(w + _RING).start()
    for w in range(_NW):
        if w + _RING < _NW:
            pred = act(w) & jnp.logical_not(act(w + _RING))
        else:
            pred = act(w)

        @pl.when(pred)
        def _(w=w):
            o_cp(w).wait()


def _route(mask, h0):
    mesh = plsc.VectorSubcoreMesh(core_axis_name="c", subcore_axis_name="s")
    compact = pl.kernel(
        _compact_body,
        mesh=mesh,
        out_type=[
            jax.ShapeDtypeStruct((_B + 16, ), jnp.int32),
            jax.ShapeDtypeStruct((2, 16), jnp.int32),
        ],
        scratch_types=[
            pltpu.VMEM((_HALF,), jnp.int32),
            pltpu.VMEM((_TPT,), jnp.int32),
            pltpu.VMEM((16,), jnp.int32),
            pltpu.SemaphoreType.DMA,
        ],
        compiler_params=pltpu.CompilerParams(needs_layout_passes=False),
    )
    idx1, cnt = compact(mask)
    gather = pl.kernel(
        _gather_body,
        mesh=mesh,
        out_type=jax.ShapeDtypeStruct((_B, _D), jnp.float32),
        scratch_types=[
            pltpu.VMEM((16,), jnp.int32),
            pltpu.VMEM((_TPT,), jnp.int32),
            pltpu.VMEM((_RING, 16, _D), jnp.float32),
        ] + [pltpu.SemaphoreType.DMA] * (2 * _RING),
        compiler_params=pltpu.CompilerParams(needs_layout_passes=False),
    )
    hc = gather(cnt, idx1, h0)
    return idx1, cnt, hc


def _stage12_body(cnt_ref, hc_ref, W1_ref, b1_ref, P1_ref, q1_ref,
                  W2_ref, b2_ref, merged_ref):
    i = pl.program_id(0)
    half = i // _HBLK
    n = cnt_ref[half, 0]
    active = (i % _HBLK) * _BM < n

    @pl.when(active)
    def _():
        hc = hc_ref[...]
        h1 = jnp.maximum(_dot(hc, W1_ref[...]) + b1_ref[...], 0.0)
        p1 = _dot(h1, P1_ref[...]) + q1_ref[...]
        m1 = _maxprob(p1) > _TAU
        f = _dot(h1, W2_ref[...]) + b2_ref[...]
        merged_ref[...] = jnp.where(m1[:, None], p1, f)


def _stage12(cnt, hc, W1, b1, P1, q1, W2, b2):
    def hc_map(i, cnt_ref):
        half = i // _HBLK
        active = (i % _HBLK) * _BM < cnt_ref[half, 0]
        return (jnp.where(active, i, 0), 0)

    full = lambda d0, d1: pl.BlockSpec((d0, d1), lambda i, c_: (0, 0))
    vec = lambda d: pl.BlockSpec((1, d), lambda i, c_: (0, 0))
    grid_spec = pltpu.PrefetchScalarGridSpec(
        num_scalar_prefetch=1,
        grid=(_NBLK,),
        in_specs=[
            pl.BlockSpec((_BM, _D), hc_map),
            full(_D, _D), vec(_D), full(_D, _C), vec(_C),
            full(_D, _C), vec(_C),
        ],
        out_specs=pl.BlockSpec((_BM, _C), lambda i, c_: (i, 0)),
    )
    return pl.pallas_call(
        _stage12_body,
        grid_spec=grid_spec,
        out_shape=jax.ShapeDtypeStruct((_B, _C), jnp.float32),
    )(cnt, hc, W1, b1.reshape(1, _D), P1, q1.reshape(1, _C),
      W2, b2.reshape(1, _C))


def _scatter_body(merged_hbm, idx_hbm, cnt_hbm, out_hbm,
                  idxchunk_v, buf_v, cbuf_v, *sems):
    rsem, wsem = sems[:_RING], sems[_RING:]
    c = lax.axis_index("c")
    s = lax.axis_index("s")
    seg = c * _HALF
    cpos0 = s * _TPT

    pltpu.sync_copy(cnt_hbm.at[c], cbuf_v)
    n = jnp.max(cbuf_v[...])
    pltpu.sync_copy(idx_hbm.at[pl.ds(seg + cpos0, _TPT)], idxchunk_v)

    def rem(w):
        return n - (cpos0 + w * 16)

    def act(w):
        return rem(w) > 0

    def full(w):
        return rem(w) >= 16

    def idvec(w):
        idxv = idxchunk_v[pl.ds(w * 16, 16)]
        return jnp.minimum(jnp.maximum(idxv, 0), jnp.int32(_B - 1))

    def r_cp(w):
        return pltpu.make_async_copy(
            merged_hbm.at[pl.ds(seg + cpos0 + w * 16, 16)],
            buf_v.at[w % _RING], rsem[w % _RING])

    def v_cp(w):
        return pltpu.make_async_copy(buf_v.at[w % _RING],
                                     out_hbm.at[idvec(w)], wsem[w % _RING])

    for w in range(min(_RING, _NW)):
        @pl.when(act(w))
        def _(w=w):
            r_cp(w).start()
    for w in range(_NW):
        @pl.when(full(w))
        def _(w=w):
            r_cp(w).wait()
            v_cp(w).start()

        @pl.when(act(w) & jnp.logical_not(full(w)))
        def _(w=w):
            r_cp(w).wait()
            idxv = idvec(w)
            lanes = lax.iota(jnp.int32, 16)

            def body(j, carry):
                t = jnp.sum(jnp.where(lanes == j, idxv, 0))
                t = jnp.minimum(jnp.maximum(t, 0), jnp.int32(_B - 1))
                pltpu.sync_copy(buf_v.at[w % _RING, pl.ds(j, 1)],
                                out_hbm.at[pl.ds(t, 1)])
                return carry

            lax.fori_loop(0, rem(w), body, jnp.int32(0))

        if w + _RING < _NW:
            @pl.when(act(w + _RING))
            def _(w=w):
                v_cp(w).wait()
                r_cp(w + _RING).start()
    for w in range(_NW):
        if w + _RING < _NW:
            pred = full(w) & jnp.logical_not(act(w + _RING))
        else:
            pred = full(w)

        @pl.when(pred)
        def _(w=w):
            v_cp(w).wait()


def _scatter(merged, idx1, cnt, out0):
    mesh = plsc.VectorSubcoreMesh(core_axis_name="c", subcore_axis_name="s")
    k = pl.kernel(
        _scatter_body,
        mesh=mesh,
        out_type=(),
        scratch_types=[
            pltpu.VMEM((_TPT,), jnp.int32),
            pltpu.VMEM((_RING, 16, _C), jnp.float32),
            pltpu.VMEM((16,), jnp.int32),
        ] + [pltpu.SemaphoreType.DMA] * (2 * _RING),
        compiler_params=pltpu.CompilerParams(needs_layout_passes=False),
    )
    out_ref = jax.new_ref(out0)
    k(merged, idx1, cnt, out_ref)
    return jax.freeze(out_ref)


def kernel(input, W0, b0, W1, b1, W2, b2, P0, q0, P1, q1):
    out0, h0, mask3d = _stage0(input, W0, b0, P0, q0)
    mask = mask3d.reshape(_B)
    idx1, cnt, hc = _route(mask, h0)
    merged = _stage12(cnt, hc, W1, b1, P1, q1, W2, b2)
    return _scatter(merged, idx1, cnt, out0)
